# baseline (speedup 1.0000x reference)
import functools

import jax
import jax.numpy as jnp
from jax import lax
from jax.experimental import pallas as pl
from jax.experimental.pallas import tpu as pltpu
from jax.experimental.pallas import tpu_sc as plsc

D = 26
VOCAB = 100000
EMB = 32
B = 4096
ZDIM = 128

NC = 2
NS = 16
NW = NC * NS
EG = EMB // 8
NTASK = D * EG
KMAX = (NTASK + NW - 1) // NW

CW = 78 * 128
VFULL = (VOCAB // CW) * CW
TAILW = VOCAB - VFULL
TAILP = 256
NCHUNK = VOCAB // CW + 1
JV = B // 16

_MESH = plsc.VectorSubcoreMesh(core_axis_name="c", subcore_axis_name="s")


def _chunk_bounds(c):
    if c < NCHUNK - 1:
        return c * CW, CW
    return VFULL, TAILW


@functools.partial(
    pl.kernel,
    out_type=jax.ShapeDtypeStruct((D * EMB, B), jnp.float32),
    mesh=_MESH,
    scratch_types=[
        pltpu.VMEM((8, CW), jnp.float32),
        pltpu.VMEM((8, B), jnp.float32),
        pltpu.VMEM((B,), jnp.int32),
        pltpu.SemaphoreType.DMA,
    ],
    compiler_params=pltpu.CompilerParams(
        use_tc_tiling_on_sc=True, needs_layout_passes=False
    ),
)
def _sc_gather(tab_hbm, tail_hbm, idx_hbm, out_hbm, buf_v, out_v, idx_v, sem):
    wid = lax.axis_index("s") * NC + lax.axis_index("c")
    lanes = lax.iota(jnp.int32, 16)

    def run_task(t):
        d = t // EG
        i = t % EG
        pltpu.sync_copy(idx_hbm.at[pl.ds(d * B, B)], idx_v)

        rows = pl.ds(pl.multiple_of(i * 8, 8), 8)

        def extract(lo, w):
            @plsc.parallel_loop(0, JV, 1, unroll=8)
            def body(j):
                v = idx_v[pl.ds(j * 16, 16)]
                col = v - lo
                m = (col >= 0) & (col < w)
                pos = j * 16 + lanes
                for e in range(8):
                    row = jnp.full((16,), e, jnp.int32)
                    g = plsc.load_gather(buf_v, [row, col], mask=m)
                    plsc.store_scatter(out_v, [row, pos], g, mask=m)

        def c_body(c, carry):
            lo = pl.multiple_of(c * CW, 128)
            pltpu.async_copy(
                tab_hbm.at[d, rows, pl.ds(lo, CW)], buf_v, sem
            ).wait()
            extract(lo, CW)
            return carry

        lax.fori_loop(0, NCHUNK - 1, c_body, 0)
        pltpu.async_copy(
            tail_hbm.at[d, rows, :], buf_v.at[:, pl.ds(0, TAILP)], sem
        ).wait()
        extract(VFULL, TAILW)
        pltpu.sync_copy(
            out_v, out_hbm.at[pl.ds(pl.multiple_of(t * 8, 8), 8), :]
        )

    def k_body(k, carry):
        t = wid + k * NW
        @pl.when(t < NTASK)
        def _():
            run_task(t)
        return carry

    lax.fori_loop(0, KMAX, k_body, 0)


def _mlp_body(xt_ref, w1_ref, b1_ref, wh_ref, bh_ref, out_ref):
    x1 = lax.dot_general(
        xt_ref[...], w1_ref[...],
        (((0,), (0,)), ((), ())),
        preferred_element_type=jnp.float32,
    )
    x1 = x1 + b1_ref[...]
    h = jnp.dot(x1, wh_ref[...], preferred_element_type=jnp.float32)
    out_ref[...] = jnp.tanh(h + bh_ref[...])


def _mlp(xt, W1, b1, Wh, bh, tb=512):
    k = D * EMB
    return pl.pallas_call(
        _mlp_body,
        grid=(B // tb,),
        in_specs=[
            pl.BlockSpec((k, tb), lambda i: (0, i)),
            pl.BlockSpec((k, 1024), lambda i: (0, 0)),
            pl.BlockSpec((1, 1024), lambda i: (0, 0)),
            pl.BlockSpec((1024, 2 * ZDIM), lambda i: (0, 0)),
            pl.BlockSpec((1, 2 * ZDIM), lambda i: (0, 0)),
        ],
        out_specs=pl.BlockSpec((tb, 2 * ZDIM), lambda i: (i, 0)),
        out_shape=jax.ShapeDtypeStruct((B, 2 * ZDIM), jnp.float32),
    )(xt, W1, b1, Wh, bh)


def kernel(x, tables, W1, b1, Wmu, bmu, Wsig, bsig):
    tab_t = jnp.transpose(tables, (0, 2, 1))
    tail = jnp.pad(tab_t[:, :, VFULL:], ((0, 0), (0, 0), (0, TAILP - TAILW)))
    x_flat = jnp.transpose(x).reshape(D * B)
    xt = _sc_gather(tab_t, tail, x_flat)
    Wh = jnp.concatenate([Wmu, Wsig], axis=1)
    bh = jnp.concatenate([bmu, bsig])[None, :]
    out = _mlp(xt, W1, b1[None, :], Wh, bh)
    return (out[:, :ZDIM], out[:, ZDIM:])


# speedup vs baseline: 1.4843x; 1.4843x over previous
import functools

import jax
import jax.numpy as jnp
from jax import lax
from jax.experimental import pallas as pl
from jax.experimental.pallas import tpu as pltpu
from jax.experimental.pallas import tpu_sc as plsc

D = 26
VOCAB = 100000
EMB = 32
B = 4096
ZDIM = 128

NC = 2
NS = 16
NW = NC * NS
EG = EMB // 8
NTASK = D * EG
KMAX = (NTASK + NW - 1) // NW

CW = 78 * 128
VFULL = (VOCAB // CW) * CW
TAILW = VOCAB - VFULL
TAILP = 256
NCHUNK = VOCAB // CW + 1
JV = B // 16

_MESH = plsc.VectorSubcoreMesh(core_axis_name="c", subcore_axis_name="s")


def _chunk_bounds(c):
    if c < NCHUNK - 1:
        return c * CW, CW
    return VFULL, TAILW


@functools.partial(
    pl.kernel,
    out_type=jax.ShapeDtypeStruct((D * EMB, B), jnp.float32),
    mesh=_MESH,
    scratch_types=[
        pltpu.VMEM((8, CW), jnp.float32),
        pltpu.VMEM((8, B), jnp.float32),
        pltpu.VMEM((B,), jnp.int32),
        pltpu.SemaphoreType.DMA,
    ],
    compiler_params=pltpu.CompilerParams(
        use_tc_tiling_on_sc=True, needs_layout_passes=False
    ),
)
def _sc_gather(tab_hbm, tail_hbm, idx_hbm, out_hbm, buf_v, out_v, idx_v, sem):
    wid = lax.axis_index("s") * NC + lax.axis_index("c")
    lanes = lax.iota(jnp.int32, 16)

    def run_task(t):
        d = t // EG
        i = t % EG
        pltpu.sync_copy(idx_hbm.at[pl.ds(d * B, B)], idx_v)

        def start(c):
            rows = pl.ds(pl.multiple_of(i * 8, 8), 8)
            if c < NCHUNK - 1:
                lo, w = _chunk_bounds(c)
                src = tab_hbm.at[d, rows, pl.ds(lo, w)]
                dst = buf_v.at[:, pl.ds(0, w)]
            else:
                src = tail_hbm.at[d, rows, :]
                dst = buf_v.at[:, pl.ds(0, TAILP)]
            return pltpu.async_copy(src, dst, sem)

        def extract(c):
            lo, w = _chunk_bounds(c)

            @plsc.parallel_loop(0, JV, 1, unroll=4)
            def body(j):
                v = idx_v[pl.ds(j * 16, 16)]
                col = v - lo
                m = (col >= 0) & (col < w)
                pos = j * 16 + lanes
                for e in range(8):
                    row = jnp.full((16,), e, jnp.int32)
                    g = plsc.load_gather(buf_v, [row, col], mask=m)
                    plsc.store_scatter(out_v, [row, pos], g, mask=m)

        for c in range(NCHUNK):
            start(c).wait()
            extract(c)
        pltpu.sync_copy(
            out_v, out_hbm.at[pl.ds(pl.multiple_of(t * 8, 8), 8), :]
        )

    for k in range(KMAX):
        t = wid + k * NW
        if k * NW + NW <= NTASK:
            run_task(t)
        else:
            @pl.when(t < NTASK)
            def _():
                run_task(t)


def _mlp_body(xt_ref, w1_ref, b1_ref, wh_ref, bh_ref, out_ref):
    x1 = lax.dot_general(
        xt_ref[...], w1_ref[...],
        (((0,), (0,)), ((), ())),
        preferred_element_type=jnp.float32,
    )
    x1 = x1 + b1_ref[...]
    h = jnp.dot(x1, wh_ref[...], preferred_element_type=jnp.float32)
    out_ref[...] = jnp.tanh(h + bh_ref[...])


def _mlp(xt, W1, b1, Wh, bh, tb=512):
    k = D * EMB
    return pl.pallas_call(
        _mlp_body,
        grid=(B // tb,),
        in_specs=[
            pl.BlockSpec((k, tb), lambda i: (0, i)),
            pl.BlockSpec((k, 1024), lambda i: (0, 0)),
            pl.BlockSpec((1, 1024), lambda i: (0, 0)),
            pl.BlockSpec((1024, 2 * ZDIM), lambda i: (0, 0)),
            pl.BlockSpec((1, 2 * ZDIM), lambda i: (0, 0)),
        ],
        out_specs=pl.BlockSpec((tb, 2 * ZDIM), lambda i: (i, 0)),
        out_shape=jax.ShapeDtypeStruct((B, 2 * ZDIM), jnp.float32),
    )(xt, W1, b1, Wh, bh)


def kernel(x, tables, W1, b1, Wmu, bmu, Wsig, bsig):
    tab_t = jnp.transpose(tables, (0, 2, 1))
    tail = jnp.pad(tab_t[:, :, VFULL:], ((0, 0), (0, 0), (0, TAILP - TAILW)))
    x_flat = jnp.transpose(x).reshape(D * B)
    xt = _sc_gather(tab_t, tail, x_flat)
    Wh = jnp.concatenate([Wmu, Wsig], axis=1)
    bh = jnp.concatenate([bmu, bsig])[None, :]
    out = _mlp(xt, W1, b1[None, :], Wh, bh)
    return (out[:, :ZDIM], out[:, ZDIM:])


# speedup vs baseline: 1.5873x; 1.0694x over previous
import functools

import jax
import jax.numpy as jnp
from jax import lax
from jax.experimental import pallas as pl
from jax.experimental.pallas import tpu as pltpu
from jax.experimental.pallas import tpu_sc as plsc

D = 26
VOCAB = 100000
EMB = 32
B = 4096
ZDIM = 128

NC = 2
NS = 16
NW = NC * NS
EG = EMB // 8
NTASK = D * EG
KMAX = (NTASK + NW - 1) // NW

CW = 88 * 128
VFULL = (VOCAB // CW) * CW
TAILW = VOCAB - VFULL
TAILA = (TAILW // 128) * 128
TS = VFULL + TAILA
TAILP = 256
NCHUNK = VOCAB // CW + 1
JV = B // 16

_MESH = plsc.VectorSubcoreMesh(core_axis_name="c", subcore_axis_name="s")


def _chunk_bounds(c):
    if c < NCHUNK - 1:
        return c * CW, CW
    return VFULL, TAILW


@functools.partial(
    pl.kernel,
    out_type=jax.ShapeDtypeStruct((D * EMB, B), jnp.float32),
    mesh=_MESH,
    scratch_types=[
        pltpu.VMEM((8, CW), jnp.float32),
        pltpu.VMEM((8, B), jnp.float32),
        pltpu.VMEM((B,), jnp.int32),
        pltpu.SemaphoreType.DMA,
    ],
    compiler_params=pltpu.CompilerParams(
        use_tc_tiling_on_sc=True, needs_layout_passes=False
    ),
)
def _sc_gather(tab_hbm, tail_hbm, idx_hbm, out_hbm, buf_v, out_v, idx_v, sem):
    wid = lax.axis_index("s") * NC + lax.axis_index("c")
    lanes = lax.iota(jnp.int32, 16)

    def run_task(t):
        d = t // EG
        i = t % EG
        pltpu.sync_copy(idx_hbm.at[pl.ds(d * B, B)], idx_v)

        rows = pl.ds(pl.multiple_of(i * 8, 8), 8)

        def start(c):
            if c < NCHUNK - 1:
                lo, w = _chunk_bounds(c)
                src = tab_hbm.at[d, rows, pl.ds(lo, w)]
                dst = buf_v.at[:, pl.ds(0, w)]
            else:
                src = tab_hbm.at[d, rows, pl.ds(VFULL, TAILA)]
                dst = buf_v.at[:, pl.ds(0, TAILA)]
            return pltpu.async_copy(src, dst, sem)

        def extract(c):
            lo, w = _chunk_bounds(c)

            @plsc.parallel_loop(0, JV, 1, unroll=4)
            def body(j):
                v = idx_v[pl.ds(j * 16, 16)]
                col = v - lo
                m = (col >= 0) & (col < w)
                pos = j * 16 + lanes
                for e in range(8):
                    row = jnp.full((16,), e, jnp.int32)
                    g = plsc.load_gather(buf_v, [row, col], mask=m)
                    plsc.store_scatter(out_v, [row, pos], g, mask=m)

        for c in range(NCHUNK):
            cp = start(c)
            if c == NCHUNK - 1:
                pltpu.async_copy(
                    tail_hbm.at[d, rows, :],
                    buf_v.at[:, pl.ds(TAILA, TAILP)], sem
                ).wait()
            cp.wait()
            extract(c)
        pltpu.sync_copy(
            out_v, out_hbm.at[pl.ds(pl.multiple_of(t * 8, 8), 8), :]
        )

    for k in range(KMAX):
        t = wid + k * NW
        if k * NW + NW <= NTASK:
            run_task(t)
        else:
            @pl.when(t < NTASK)
            def _():
                run_task(t)


def _mlp_body(xt_ref, w1_ref, b1_ref, wh_ref, bh_ref, out_ref):
    x1 = lax.dot_general(
        xt_ref[...], w1_ref[...],
        (((0,), (0,)), ((), ())),
        preferred_element_type=jnp.float32,
    )
    x1 = x1 + b1_ref[...]
    h = jnp.dot(x1, wh_ref[...], preferred_element_type=jnp.float32)
    out_ref[...] = jnp.tanh(h + bh_ref[...])


def _mlp(xt, W1, b1, Wh, bh, tb=512):
    k = D * EMB
    return pl.pallas_call(
        _mlp_body,
        grid=(B // tb,),
        in_specs=[
            pl.BlockSpec((k, tb), lambda i: (0, i)),
            pl.BlockSpec((k, 1024), lambda i: (0, 0)),
            pl.BlockSpec((1, 1024), lambda i: (0, 0)),
            pl.BlockSpec((1024, 2 * ZDIM), lambda i: (0, 0)),
            pl.BlockSpec((1, 2 * ZDIM), lambda i: (0, 0)),
        ],
        out_specs=pl.BlockSpec((tb, 2 * ZDIM), lambda i: (i, 0)),
        out_shape=jax.ShapeDtypeStruct((B, 2 * ZDIM), jnp.float32),
    )(xt, W1, b1, Wh, bh)


def kernel(x, tables, W1, b1, Wmu, bmu, Wsig, bsig):
    tab_t = jnp.transpose(tables, (0, 2, 1))
    tail = jnp.pad(tab_t[:, :, TS:], ((0, 0), (0, 0), (0, TAILP - (VOCAB - TS))))
    x_flat = jnp.transpose(x).reshape(D * B)
    xt = _sc_gather(tab_t, tail, x_flat)
    Wh = jnp.concatenate([Wmu, Wsig], axis=1)
    bh = jnp.concatenate([bmu, bsig])[None, :]
    out = _mlp(xt, W1, b1[None, :], Wh, bh)
    return (out[:, :ZDIM], out[:, ZDIM:])
